# baseline (speedup 1.0000x reference)
import functools

import jax
import jax.numpy as jnp
from jax import lax
from jax.experimental import pallas as pl
from jax.experimental.pallas import tpu as pltpu
from jax.experimental.pallas import tpu_sc as plsc

NC = 2
NS = 16
NW = NC * NS

GC = 128
CC = 64


def _body(n_g, n_c, ctx_sz, d,
          idx_hbm, ctx_hbm, tab_hbm, out_hbm,
          idx_v, rep_v, rows_v, sem, rep_sem):
  cid = lax.axis_index("c")
  sid = lax.axis_index("s")
  wid = sid * NC + cid
  b_per_w = n_g * GC
  base = wid * b_per_w

  pltpu.sync_copy(idx_hbm.at[wid], idx_v)

  rep_cps = [pltpu.async_copy(ctx_hbm, rep_v.at[r], rep_sem)
             for r in range(CC)]

  for g in range(n_g):
    pltpu.async_copy(tab_hbm.at[idx_v.at[g]], rows_v, sem).wait()
    pltpu.sync_copy(
        rows_v, out_hbm.at[pl.ds(base + g * GC, GC), pl.ds(ctx_sz, d)])

  for cp in rep_cps:
    cp.wait()

  for t in range(n_c):
    pltpu.sync_copy(
        rep_v, out_hbm.at[pl.ds(base + t * CC, CC), pl.ds(0, ctx_sz)])


def kernel(class_indices, ctx, class_embeddings):
  B = class_indices.shape[0]
  n_ctx, d = ctx.shape
  ctx_sz = n_ctx * d
  b_per_w = B // NW
  n_g = b_per_w // GC
  n_c = b_per_w // CC

  idx_r = class_indices.reshape(NW, n_g, GC).astype(jnp.int32)
  ctx_r = ctx.reshape(ctx_sz)

  mesh = plsc.VectorSubcoreMesh(core_axis_name="c", subcore_axis_name="s")
  run = pl.kernel(
      functools.partial(_body, n_g, n_c, ctx_sz, d),
      out_type=jax.ShapeDtypeStruct((B, ctx_sz + d), jnp.float32),
      mesh=mesh,
      compiler_params=pltpu.CompilerParams(use_tc_tiling_on_sc=False),
      scratch_types=[
          pltpu.VMEM((n_g, GC), jnp.int32),
          pltpu.VMEM((CC, ctx_sz), jnp.float32),
          pltpu.VMEM((GC, d), jnp.float32),
          pltpu.SemaphoreType.DMA,
          pltpu.SemaphoreType.DMA,
      ],
  )
  out2 = run(idx_r, ctx_r, class_embeddings)
  return out2.reshape(B, n_ctx + 1, d)


# speedup vs baseline: 1.2151x; 1.2151x over previous
import functools

import jax
import jax.numpy as jnp
from jax import lax
from jax.experimental import pallas as pl
from jax.experimental.pallas import tpu as pltpu
from jax.experimental.pallas import tpu_sc as plsc

NC = 2
NS = 16
NW = NC * NS

GC = 128
LB = 2048


def _g1_body(n_g, d, idx_hbm, tab_hbm, emb_hbm, idx_v, rows_v, sem):
  cid = lax.axis_index("c")
  sid = lax.axis_index("s")
  wid = sid * NC + cid
  base = wid * (n_g * GC)
  pltpu.sync_copy(idx_hbm.at[wid], idx_v)
  for g in range(n_g):
    pltpu.async_copy(tab_hbm.at[idx_v.at[g]], rows_v, sem).wait()
    pltpu.sync_copy(rows_v, emb_hbm.at[pl.ds(base + g * GC, GC)])


def _t1_body(ctx_ref, out_ref):
  s = pl.program_id(0)
  n = ctx_ref.shape[1]
  oh = (lax.broadcasted_iota(jnp.int32, (n, 1), 0) == s).astype(jnp.float32)
  col = jnp.dot(ctx_ref[...], oh, preferred_element_type=jnp.float32)
  out_ref[...] = jnp.broadcast_to(col, (64, LB))[None]


def _t2_body(_, emb_ref, out_ref):
  eye = (lax.broadcasted_iota(jnp.int32, (64, 64), 0) ==
         lax.broadcasted_iota(jnp.int32, (64, 64), 1)).astype(jnp.float32)
  out_ref[...] = lax.dot_general(
      eye, emb_ref[...], (((1,), (1,)), ((), ())),
      preferred_element_type=jnp.float32)[None]


def kernel(class_indices, ctx, class_embeddings):
  B = class_indices.shape[0]
  n_ctx, d = ctx.shape
  b_per_w = B // NW
  n_g = b_per_w // GC

  idx_r = class_indices.reshape(NW, n_g, GC).astype(jnp.int32)

  mesh = plsc.VectorSubcoreMesh(core_axis_name="c", subcore_axis_name="s")
  emb = pl.kernel(
      functools.partial(_g1_body, n_g, d),
      out_type=jax.ShapeDtypeStruct((B, d), jnp.float32),
      mesh=mesh,
      compiler_params=pltpu.CompilerParams(use_tc_tiling_on_sc=False),
      scratch_types=[
          pltpu.VMEM((n_g, GC), jnp.int32),
          pltpu.VMEM((GC, d), jnp.float32),
          pltpu.SemaphoreType.DMA,
      ],
  )(idx_r, class_embeddings)

  ctx_t = ctx.T
  out17 = pl.pallas_call(
      _t1_body,
      grid=(n_ctx, B // LB),
      in_specs=[pl.BlockSpec((d, n_ctx), lambda s, t: (0, 0))],
      out_specs=pl.BlockSpec((1, d, LB), lambda s, t: (s, 0, t)),
      out_shape=jax.ShapeDtypeStruct((n_ctx + 1, d, B), jnp.float32),
  )(ctx_t)

  out17 = pl.pallas_call(
      _t2_body,
      grid=(B // LB,),
      in_specs=[
          pl.BlockSpec(memory_space=pltpu.MemorySpace.HBM),
          pl.BlockSpec((LB, d), lambda t: (t, 0)),
      ],
      out_specs=pl.BlockSpec((1, d, LB), lambda t: (n_ctx, 0, t)),
      out_shape=jax.ShapeDtypeStruct((n_ctx + 1, d, B), jnp.float32),
      input_output_aliases={0: 0},
  )(out17, emb)

  return out17.transpose(2, 0, 1)
